# Initial kernel scaffold; baseline (speedup 1.0000x reference)
import jax
import jax.numpy as jnp
from jax.experimental import pallas as pl


def kernel(hidden_states, sim_matrix, gates, q_proj, k_proj, v_proj, o_proj):
    raise NotImplementedError("write your pallas kernel here")


# speedup vs baseline: 13.3960x; 13.3960x over previous
import functools

import jax
import jax.numpy as jnp
from jax.experimental import pallas as pl

B, T, C, H, E = 2, 512, 768, 64, 16
N = B * T
EH = E * H
NEG = -1e9
F32_MIN = -3.0e38


def _dyn_smha_kernel(flat_ref, sim_ref, gates_ref, wqkv_ref, wo_ref, out_ref):
    flat = flat_ref[...]
    sim = sim_ref[...]
    gates = gates_ref[...]

    fnorm = jnp.sqrt(jnp.sum(flat * flat, axis=1, keepdims=True))
    fn = flat / jnp.maximum(fnorm, 1e-12)
    snorm = jnp.sqrt(jnp.sum(sim * sim, axis=0, keepdims=True))
    sn = sim / jnp.maximum(snorm, 1e-12)
    logits = jnp.dot(fn, sn, preferred_element_type=jnp.float32)
    logits = logits - jax.nn.sigmoid(gates)

    gated = jnp.maximum(logits, 0.0)
    act = (gated > 0.0).astype(jnp.float32)
    inactive = jnp.sum(act, axis=1, keepdims=True) == 0.0

    eidx = jax.lax.broadcasted_iota(jnp.int32, (N, E), 1)
    m1 = jnp.max(logits, axis=1, keepdims=True)
    i1 = jnp.min(jnp.where(logits == m1, eidx, E), axis=1, keepdims=True)
    first = eidx == i1
    l2 = jnp.where(first, F32_MIN, logits)
    m2 = jnp.max(l2, axis=1, keepdims=True)
    i2 = jnp.min(jnp.where(l2 == m2, eidx, E), axis=1, keepdims=True)
    fb = jnp.logical_or(first, eidx == i2).astype(jnp.float32)

    mask = jnp.where(inactive, fb, act)
    masked = jnp.where(mask > 0.0, gated, NEG)
    mmax = jnp.max(masked, axis=1, keepdims=True)
    p = jnp.exp(masked - mmax)
    w = p / jnp.sum(p, axis=1, keepdims=True)
    w = w * mask

    qkv = jnp.dot(flat, wqkv_ref[...], preferred_element_type=jnp.float32)

    def combine(base):
        acc = w[:, 0:1] * qkv[:, base:base + H]
        for e in range(1, E):
            acc = acc + w[:, e:e + 1] * qkv[:, base + e * H:base + (e + 1) * H]
        return acc

    q = combine(0)
    k = combine(EH)
    v = combine(2 * EH)

    scale = 1.0 / (H ** 0.5)
    row = jax.lax.broadcasted_iota(jnp.int32, (T, T), 0)
    col = jax.lax.broadcasted_iota(jnp.int32, (T, T), 1)
    causal = row >= col

    attn_out_parts = []
    for b in range(B):
        qb = q[b * T:(b + 1) * T, :]
        kb = k[b * T:(b + 1) * T, :]
        vb = v[b * T:(b + 1) * T, :]
        scores = jax.lax.dot_general(
            qb, kb, (((1,), (1,)), ((), ())),
            preferred_element_type=jnp.float32) * scale
        scores = jnp.where(causal, scores, NEG)
        smax = jnp.max(scores, axis=1, keepdims=True)
        sp = jnp.exp(scores - smax)
        attn = sp / jnp.sum(sp, axis=1, keepdims=True)
        attn_out_parts.append(
            jnp.dot(attn, vb, preferred_element_type=jnp.float32))
    a = jnp.concatenate(attn_out_parts, axis=0)

    aw = jnp.concatenate([w[:, e:e + 1] * a for e in range(E)], axis=1)
    out_ref[...] = jnp.dot(aw, wo_ref[...], preferred_element_type=jnp.float32)


@functools.partial(jax.jit, static_argnames=("interpret",))
def kernel(hidden_states, sim_matrix, gates, q_proj, k_proj, v_proj, o_proj,
           interpret=False):
    flat = hidden_states.reshape(N, C)
    wqkv = jnp.concatenate(
        [p.transpose(1, 0, 2).reshape(C, EH) for p in (q_proj, k_proj, v_proj)],
        axis=1)
    wo = o_proj.reshape(EH, C)
    gates2 = gates.reshape(1, E)

    out = pl.pallas_call(
        _dyn_smha_kernel,
        out_shape=jax.ShapeDtypeStruct((N, C), jnp.float32),
        interpret=interpret,
    )(flat, sim_matrix, gates2, wqkv, wo)
    return out.reshape(B, T, C)


# speedup vs baseline: 20.4365x; 1.5256x over previous
import functools

import jax
import jax.numpy as jnp
from jax.experimental import pallas as pl

B, T, C, H, E = 2, 512, 768, 64, 16
N = B * T
EH = E * H
NEG = -1e9
F32_MIN = -3.0e38


def _dyn_smha_kernel(flat_ref, simt_ref, gates_ref, wqkvt_ref, wo_ref, out_ref):
    flat = flat_ref[...]
    simt = simt_ref[...]
    gates = gates_ref[...]

    fsq = jnp.sum(flat * flat, axis=1, keepdims=True)
    fn = flat / jnp.maximum(jnp.sqrt(fsq), 1e-12)
    ssq = jnp.sum(simt * simt, axis=1, keepdims=True)
    sn = simt / jnp.maximum(jnp.sqrt(ssq), 1e-12)
    logits = jax.lax.dot_general(
        sn, fn, (((1,), (1,)), ((), ())),
        preferred_element_type=jnp.float32)
    logits = logits - jax.nn.sigmoid(gates)

    gated = jnp.maximum(logits, 0.0)
    act = (gated > 0.0).astype(jnp.float32)
    inactive = jnp.sum(act, axis=0, keepdims=True) == 0.0

    eidx = jax.lax.broadcasted_iota(jnp.int32, (E, N), 0)
    m1 = jnp.max(logits, axis=0, keepdims=True)
    i1 = jnp.min(jnp.where(logits == m1, eidx, E), axis=0, keepdims=True)
    first = eidx == i1
    l2 = jnp.where(first, F32_MIN, logits)
    m2 = jnp.max(l2, axis=0, keepdims=True)
    i2 = jnp.min(jnp.where(l2 == m2, eidx, E), axis=0, keepdims=True)
    fb = jnp.logical_or(first, eidx == i2).astype(jnp.float32)

    mask = jnp.where(inactive, fb, act)
    masked = jnp.where(mask > 0.0, gated, NEG)
    mmax = jnp.max(masked, axis=0, keepdims=True)
    p = jnp.exp(masked - mmax)
    w = p / jnp.sum(p, axis=0, keepdims=True)
    w = w * mask

    qkvt = jax.lax.dot_general(
        wqkvt_ref[...], flat, (((1,), (1,)), ((), ())),
        preferred_element_type=jnp.float32)

    def combine(base):
        acc = w[0:1, :] * qkvt[base:base + H, :]
        for e in range(1, E):
            acc = acc + w[e:e + 1, :] * qkvt[base + e * H:base + (e + 1) * H, :]
        return acc

    qt = combine(0)
    kt = combine(EH)
    vt = combine(2 * EH)

    scale = 1.0 / (H ** 0.5)
    row = jax.lax.broadcasted_iota(jnp.int32, (T, T), 0)
    col = jax.lax.broadcasted_iota(jnp.int32, (T, T), 1)
    causal = row >= col

    at_parts = []
    for b in range(B):
        qb = qt[:, b * T:(b + 1) * T]
        kb = kt[:, b * T:(b + 1) * T]
        vb = vt[:, b * T:(b + 1) * T]
        scores = jax.lax.dot_general(
            qb, kb, (((0,), (0,)), ((), ())),
            preferred_element_type=jnp.float32) * scale
        scores = jnp.where(causal, scores, NEG)
        smax = jnp.max(scores, axis=1, keepdims=True)
        sp = jnp.exp(scores - smax)
        attn = sp / jnp.sum(sp, axis=1, keepdims=True)
        at_parts.append(jax.lax.dot_general(
            vb, attn, (((1,), (1,)), ((), ())),
            preferred_element_type=jnp.float32))
    at = jnp.concatenate(at_parts, axis=1)

    awt = jnp.concatenate([at * w[e:e + 1, :] for e in range(E)], axis=0)
    out_ref[...] = jax.lax.dot_general(
        awt, wo_ref[...], (((0,), (0,)), ((), ())),
        preferred_element_type=jnp.float32)


@functools.partial(jax.jit, static_argnames=("interpret",))
def kernel(hidden_states, sim_matrix, gates, q_proj, k_proj, v_proj, o_proj,
           interpret=False):
    flat = hidden_states.reshape(N, C)
    wqkvt = jnp.concatenate(
        [p.transpose(0, 2, 1).reshape(EH, C) for p in (q_proj, k_proj, v_proj)],
        axis=0)
    wo = o_proj.reshape(EH, C)
    simt = sim_matrix.T
    gates2 = gates.reshape(E, 1)

    out = pl.pallas_call(
        _dyn_smha_kernel,
        out_shape=jax.ShapeDtypeStruct((N, C), jnp.float32),
        interpret=interpret,
    )(flat, simt, gates2, wqkvt, wo)
    return out.reshape(B, T, C)


# speedup vs baseline: 21.0934x; 1.0321x over previous
import functools

import jax
import jax.numpy as jnp
from jax.experimental import pallas as pl

B, T, C, H, E = 2, 512, 768, 64, 16
N = B * T
EH = E * H
NEG = -1e9
F32_MIN = -3.0e38


def _dyn_smha_kernel(flat_ref, simt_ref, gates_ref, wqkvt_ref, wo_ref, out_ref):
    flat = flat_ref[...]
    simt = simt_ref[...]
    gates = gates_ref[...]

    fsq = jnp.sum(flat * flat, axis=1, keepdims=True)
    fn = flat / jnp.maximum(jnp.sqrt(fsq), 1e-12)
    ssq = jnp.sum(simt * simt, axis=1, keepdims=True)
    sn = simt / jnp.maximum(jnp.sqrt(ssq), 1e-12)
    logits = jax.lax.dot_general(
        sn, fn, (((1,), (1,)), ((), ())),
        preferred_element_type=jnp.float32)
    logits = logits - jax.nn.sigmoid(gates)

    gated = jnp.maximum(logits, 0.0)
    act = (gated > 0.0).astype(jnp.float32)
    inactive = jnp.sum(act, axis=0, keepdims=True) == 0.0

    eidx = jax.lax.broadcasted_iota(jnp.int32, (E, N), 0)
    m1 = jnp.max(logits, axis=0, keepdims=True)
    i1 = jnp.min(jnp.where(logits == m1, eidx, E), axis=0, keepdims=True)
    first = eidx == i1
    l2 = jnp.where(first, F32_MIN, logits)
    m2 = jnp.max(l2, axis=0, keepdims=True)
    i2 = jnp.min(jnp.where(l2 == m2, eidx, E), axis=0, keepdims=True)
    fb = jnp.logical_or(first, eidx == i2).astype(jnp.float32)

    mask = jnp.where(inactive, fb, act)
    masked = jnp.where(mask > 0.0, gated, NEG)
    mmax = jnp.max(masked, axis=0, keepdims=True)
    p = jnp.exp(masked - mmax)
    w = p / jnp.sum(p, axis=0, keepdims=True)
    w = w * mask

    qkvt = jax.lax.dot_general(
        wqkvt_ref[...], flat.astype(jnp.bfloat16), (((1,), (1,)), ((), ())),
        preferred_element_type=jnp.float32)

    def combine(base):
        acc = w[0:1, :] * qkvt[base:base + H, :]
        for e in range(1, E):
            acc = acc + w[e:e + 1, :] * qkvt[base + e * H:base + (e + 1) * H, :]
        return acc

    qt = combine(0)
    kt = combine(EH)
    vt = combine(2 * EH)

    scale = 1.0 / (H ** 0.5)
    row = jax.lax.broadcasted_iota(jnp.int32, (T, T), 0)
    col = jax.lax.broadcasted_iota(jnp.int32, (T, T), 1)
    causal = row >= col

    at_parts = []
    for b in range(B):
        qb = qt[:, b * T:(b + 1) * T].astype(jnp.bfloat16)
        kb = kt[:, b * T:(b + 1) * T].astype(jnp.bfloat16)
        vb = vt[:, b * T:(b + 1) * T].astype(jnp.bfloat16)
        scores = jax.lax.dot_general(
            qb, kb, (((0,), (0,)), ((), ())),
            preferred_element_type=jnp.float32) * scale
        scores = jnp.where(causal, scores, NEG)
        smax = jnp.max(scores, axis=1, keepdims=True)
        sp = jnp.exp(scores - smax)
        attn = sp / jnp.sum(sp, axis=1, keepdims=True)
        at_parts.append(jax.lax.dot_general(
            vb, attn.astype(jnp.bfloat16), (((1,), (1,)), ((), ())),
            preferred_element_type=jnp.float32))
    at = jnp.concatenate(at_parts, axis=1)

    awt = jnp.concatenate(
        [(at * w[e:e + 1, :]).astype(jnp.bfloat16) for e in range(E)], axis=0)
    out_ref[...] = jax.lax.dot_general(
        awt, wo_ref[...], (((0,), (0,)), ((), ())),
        preferred_element_type=jnp.float32)


@functools.partial(jax.jit, static_argnames=("interpret",))
def kernel(hidden_states, sim_matrix, gates, q_proj, k_proj, v_proj, o_proj,
           interpret=False):
    flat = hidden_states.reshape(N, C)
    wqkvt = jnp.concatenate(
        [p.transpose(0, 2, 1).reshape(EH, C) for p in (q_proj, k_proj, v_proj)],
        axis=0).astype(jnp.bfloat16)
    wo = o_proj.reshape(EH, C).astype(jnp.bfloat16)
    simt = sim_matrix.T
    gates2 = gates.reshape(E, 1)

    out = pl.pallas_call(
        _dyn_smha_kernel,
        out_shape=jax.ShapeDtypeStruct((N, C), jnp.float32),
        interpret=interpret,
    )(flat, simt, gates2, wqkvt, wo)
    return out.reshape(B, T, C)
